# baseline (speedup 1.0000x reference)
import functools

import jax
import jax.numpy as jnp
from jax import lax
from jax.experimental import pallas as pl
from jax.experimental.pallas import tpu as pltpu
from jax.experimental.pallas import tpu_sc as plsc

F32 = jnp.float32
I32 = jnp.int32
BF = jnp.bfloat16

NC = 2
NS = 16
NW = NC * NS
CH = 128


def _sc_mesh():
    return plsc.VectorSubcoreMesh(
        core_axis_name="c", subcore_axis_name="s", num_cores=NC, num_subcores=NS)


def _prep(selection, sel_w, sel_b, w1u, b1, wn2u, bn2):
    G = selection.shape[0]
    F1 = w1u.shape[1]
    F2 = wn2u.shape[1]

    def body(sel_ref, sw_ref, sb_ref, w1u_ref, b1_ref, wn2u_ref, bn2_ref,
             uw1_ref, un_ref):
        u = jnp.dot(sel_ref[...], sw_ref[...], preferred_element_type=F32)
        u = u + sb_ref[...]
        uw1_ref[...] = jnp.dot(u, w1u_ref[...], preferred_element_type=F32) + b1_ref[...]
        un_ref[...] = jnp.dot(u, wn2u_ref[...], preferred_element_type=F32) + bn2_ref[...]

    return pl.pallas_call(
        body,
        out_shape=(jax.ShapeDtypeStruct((G, F1), F32),
                   jax.ShapeDtypeStruct((G, F2), F32)),
    )(selection, sel_w, sel_b, w1u, b1, wn2u, bn2)


def _gather(tpad, row2d, col2d, E):
    nchunk = E // CH

    @functools.partial(
        pl.kernel,
        out_type=(jax.ShapeDtypeStruct((E, 16), F32),
                  jax.ShapeDtypeStruct((E, 16), F32)),
        mesh=_sc_mesh(),
        scratch_types=(
            pltpu.VMEM((CH,), I32),
            pltpu.VMEM((CH,), I32),
            pltpu.VMEM((CH, 16), F32),
            pltpu.VMEM((CH, 16), F32),
            pltpu.SemaphoreType.DMA,
            pltpu.SemaphoreType.DMA,
        ),
        compiler_params=pltpu.CompilerParams(use_tc_tiling_on_sc=False),
    )
    def k(t_hbm, r_hbm, c_hbm, gr_hbm, gc_hbm, ir_v, ic_v, br_v, bc_v, sr, sc_):
        cid = lax.axis_index("c")
        sid = lax.axis_index("s")
        wid = sid * NC + cid
        nloc = (nchunk - wid + NW - 1) // NW

        def step(j, carry):
            ch = wid + j * NW
            pltpu.sync_copy(r_hbm.at[ch], ir_v)
            pltpu.sync_copy(c_hbm.at[ch], ic_v)
            d1 = pltpu.async_copy(t_hbm.at[ir_v], br_v, sr)
            d2 = pltpu.async_copy(t_hbm.at[ic_v], bc_v, sc_)
            d1.wait()
            d2.wait()
            pltpu.sync_copy(br_v, gr_hbm.at[pl.ds(ch * CH, CH)])
            pltpu.sync_copy(bc_v, gc_hbm.at[pl.ds(ch * CH, CH)])
            return carry

        lax.fori_loop(0, nloc, step, 0)

    return k(tpad, row2d, col2d)


def _edge(gr, gc, ea, uw1, p16, q16, w1c, w2, b2, w3, b3, w4, b4, w5, b5,
          v1a, v1b, vb1, v2, vb2, B):
    E = gr.shape[0]
    nb = E // B
    G = uw1.shape[0]

    def body(gr_ref, gc_ref, ea_ref, uw1_ref, p_ref, q_ref, w1c_ref,
             w2_ref, b2_ref, w3_ref, b3_ref, w4_ref, b4_ref, w5_ref, b5_ref,
             v1a_ref, v1b_ref, vb1_ref, v2_ref, vb2_ref,
             o0_ref, o1_ref, o2_ref, o3_ref):
        HB = B // 2
        for s0 in (0, HB):
            g = gr_ref[pl.ds(s0, HB), :]
            gcv = gc_ref[pl.ds(s0, HB), :]
            ea = ea_ref[pl.ds(s0, HB), :]
            oh = (g[:, 9:10].astype(I32)
                  == lax.broadcasted_iota(I32, (HB, G), 1)).astype(F32)
            h = jnp.dot(g, p_ref[...], preferred_element_type=F32)
            h = h + jnp.dot(gcv, q_ref[...], preferred_element_type=F32)
            h = h + jnp.dot(oh, uw1_ref[...], preferred_element_type=F32)
            h = h + ea * w1c_ref[...]
            h = jnp.maximum(h, 0.0)
            h = jnp.maximum(jnp.dot(h, w2_ref[...], preferred_element_type=F32) + b2_ref[...], 0.0)
            h = jnp.maximum(jnp.dot(h, w3_ref[...], preferred_element_type=F32) + b3_ref[...], 0.0)
            h = jnp.maximum(jnp.dot(h, w4_ref[...], preferred_element_type=F32) + b4_ref[...], 0.0)
            e5 = jnp.dot(h, w5_ref[...], preferred_element_type=F32) + b5_ref[...]
            m = jnp.dot(gcv, v1a_ref[...], preferred_element_type=F32)
            m = m + jnp.dot(e5, v1b_ref[...], preferred_element_type=F32) + vb1_ref[...]
            m = jnp.maximum(m, 0.0)
            m = jnp.maximum(jnp.dot(m, v2_ref[...], preferred_element_type=F32) + vb2_ref[...], 0.0)
            o0_ref[pl.ds(s0, HB), :] = m[:, 0:128]
            o1_ref[pl.ds(s0, HB), :] = m[:, 128:256]
            o2_ref[pl.ds(s0, HB), :] = m[:, 256:384]
            o3_ref[pl.ds(s0, HB), :] = m[:, 384:512]

    def cst(*s):
        return pl.BlockSpec(s, lambda i: tuple(0 for _ in s))

    return pl.pallas_call(
        body,
        grid=(nb,),
        in_specs=[
            pl.BlockSpec((B, 16), lambda i: (i, 0)),
            pl.BlockSpec((B, 16), lambda i: (i, 0)),
            pl.BlockSpec((B, 1), lambda i: (i, 0)),
            cst(*uw1.shape), cst(*p16.shape), cst(*q16.shape), cst(*w1c.shape),
            cst(*w2.shape), cst(*b2.shape), cst(*w3.shape), cst(*b3.shape),
            cst(*w4.shape), cst(*b4.shape), cst(*w5.shape), cst(*b5.shape),
            cst(*v1a.shape), cst(*v1b.shape), cst(*vb1.shape),
            cst(*v2.shape), cst(*vb2.shape),
        ],
        out_specs=[pl.BlockSpec((B, 128), lambda i: (i, 0))] * 4,
        out_shape=[jax.ShapeDtypeStruct((E, 128), F32)] * 4,
    )(gr, gc, ea, uw1, p16, q16, w1c, w2, b2, w3, b3, w4, b4, w5, b5,
      v1a, v1b, vb1, v2, vb2)


def _scatter(m4, row2d, zer_cst, E, n_pad):
    nchunk = E // CH
    FC = 64
    stripe = n_pad // NS

    @functools.partial(
        pl.kernel,
        out_type=jax.ShapeDtypeStruct((NC, n_pad, 512), F32),
        mesh=_sc_mesh(),
        scratch_types=(
            pltpu.VMEM((2, CH), I32),
            pltpu.VMEM((CH, FC), F32),
            pltpu.VMEM((CH, FC), F32),
            pltpu.VMEM((stripe, FC), F32),
            pltpu.VMEM_SHARED((n_pad, FC), F32),
            pltpu.SemaphoreType.DMA,
            pltpu.SemaphoreType.DMA,
            pltpu.SemaphoreType.DMA,
            pltpu.SemaphoreType.DMA,
        ),
        compiler_params=pltpu.CompilerParams(use_tc_tiling_on_sc=False),
    )
    def k(m0_hbm, m1_hbm, m2_hbm, m3_hbm, r_hbm, zer_hbm, sp_hbm,
          idx_v, mv0, mv1, zer_v, s_sh, si0, si1, sm0, sm1):
        cid = lax.axis_index("c")
        sid = lax.axis_index("s")
        wid = sid * NC + cid
        nloc = (nchunk - wid + NW - 1) // NW
        pltpu.sync_copy(zer_hbm, zer_v)
        mvs = (mv0, mv1)
        sis = (si0, si1)
        sms = (sm0, sm1)
        for p in range(8):
            src = (m0_hbm, m1_hbm, m2_hbm, m3_hbm)[p // 2]
            c0 = (p % 2) * FC
            f0 = p * FC
            pltpu.sync_copy(zer_v, s_sh.at[pl.ds(sid * stripe, stripe)])
            plsc.subcore_barrier()

            def start(j, b, src=src, c0=c0):
                ch = wid + j * NW
                pltpu.async_copy(r_hbm.at[ch], idx_v.at[b], sis[b])
                pltpu.async_copy(
                    src.at[pl.ds(ch * CH, CH), pl.ds(c0, FC)], mvs[b], sms[b])

            def drain(b, src=src, c0=c0):
                pltpu.make_async_copy(r_hbm.at[0], idx_v.at[b], sis[b]).wait()
                pltpu.make_async_copy(
                    src.at[pl.ds(0, CH), pl.ds(c0, FC)], mvs[b], sms[b]).wait()

            @pl.when(nloc > 0)
            def _():
                start(0, 0)

            def step(j, carry):
                even = lax.rem(j, 2) == 0

                @pl.when(even)
                def _():
                    drain(0)
                    @pl.when(j + 1 < nloc)
                    def _():
                        start(j + 1, 1)
                    pltpu.sync_copy(mv0, s_sh.at[idx_v.at[0]], add=True)

                @pl.when(jnp.logical_not(even))
                def _():
                    drain(1)
                    @pl.when(j + 1 < nloc)
                    def _():
                        start(j + 1, 0)
                    pltpu.sync_copy(mv1, s_sh.at[idx_v.at[1]], add=True)

                return carry

            lax.fori_loop(0, nloc, step, 0)
            plsc.subcore_barrier()
            pltpu.sync_copy(
                s_sh.at[pl.ds(sid * stripe, stripe)],
                sp_hbm.at[cid, pl.ds(sid * stripe, stripe), pl.ds(f0, FC)])
            plsc.subcore_barrier()

    return k(m4[0], m4[1], m4[2], m4[3], row2d, zer_cst)


def _count(row2d, ones_cst, zer_cst, E, n_pad):
    nchunk = E // CH
    stripe = n_pad // NS

    @functools.partial(
        pl.kernel,
        out_type=jax.ShapeDtypeStruct((NC, n_pad, 16), F32),
        mesh=_sc_mesh(),
        scratch_types=(
            pltpu.VMEM((1, CH), I32),
            pltpu.VMEM((CH, 16), F32),
            pltpu.VMEM((stripe, 16), F32),
            pltpu.VMEM_SHARED((n_pad, 16), F32),
        ),
        compiler_params=pltpu.CompilerParams(use_tc_tiling_on_sc=False),
    )
    def k(r_hbm, ones_hbm, zer_hbm, cp_hbm, idx_v, ones_v, zer_v, c_sh):
        cid = lax.axis_index("c")
        sid = lax.axis_index("s")
        wid = sid * NC + cid
        nloc = (nchunk - wid + NW - 1) // NW
        pltpu.sync_copy(ones_hbm, ones_v)
        pltpu.sync_copy(zer_hbm, zer_v)
        pltpu.sync_copy(zer_v, c_sh.at[pl.ds(sid * stripe, stripe)])
        plsc.subcore_barrier()

        def step(j, carry):
            ch = wid + j * NW
            pltpu.sync_copy(r_hbm.at[ch], idx_v.at[0])
            pltpu.sync_copy(ones_v, c_sh.at[idx_v.at[0]], add=True)
            return carry

        lax.fori_loop(0, nloc, step, 0)
        plsc.subcore_barrier()
        pltpu.sync_copy(c_sh.at[pl.ds(sid * stripe, stripe)],
                        cp_hbm.at[cid, pl.ds(sid * stripe, stripe)])

    return k(row2d, ones_cst, zer_cst)


def _node(tpad, sps, cp, un, p2a, wb, w2row, b2, BN):
    n_pad = tpad.shape[0]
    nb = n_pad // BN
    G = un.shape[0]

    def body(t_ref, s0_ref, s1_ref, s2_ref, s3_ref, cp_ref, un_ref, p2_ref,
             wb_ref, w2_ref, b2_ref, o_ref):
        t = t_ref[...]
        cpv = cp_ref[...]
        s = None
        for r in (s0_ref, s1_ref, s2_ref, s3_ref):
            v = r[...]
            vv = v[0] + v[1]
            s = vv if s is None else s + vv
        cnt = cpv[0][:, 0:1] + cpv[1][:, 0:1]
        mean = s / jnp.maximum(cnt, 1.0)
        oh = (t[:, 9:10].astype(I32)
              == lax.broadcasted_iota(I32, (BN, G), 1)).astype(F32)
        l1 = jnp.dot(t, p2_ref[...], preferred_element_type=F32)
        l1 = l1 + jnp.dot(mean, wb_ref[...], preferred_element_type=F32)
        l1 = l1 + jnp.dot(oh, un_ref[...], preferred_element_type=F32)
        l1 = jnp.maximum(l1, 0.0)
        o_ref[...] = jnp.sum(l1 * w2_ref[...], axis=1, keepdims=True) + b2_ref[...]

    def cst(*s):
        return pl.BlockSpec(s, lambda i: tuple(0 for _ in s))

    return pl.pallas_call(
        body,
        grid=(nb,),
        in_specs=[
            pl.BlockSpec((BN, 16), lambda i: (i, 0)),
            pl.BlockSpec((2, BN, 512), lambda i: (0, i, 0)),
            pl.BlockSpec((2, BN, 512), lambda i: (0, i, 0)),
            pl.BlockSpec((2, BN, 512), lambda i: (0, i, 0)),
            pl.BlockSpec((2, BN, 512), lambda i: (0, i, 0)),
            pl.BlockSpec((2, BN, 16), lambda i: (0, i, 0)),
            cst(*un.shape), cst(*p2a.shape), cst(*wb.shape),
            cst(*w2row.shape), cst(*b2.shape),
        ],
        out_specs=pl.BlockSpec((BN, 1), lambda i: (i, 0)),
        out_shape=jax.ShapeDtypeStruct((n_pad, 1), F32),
    )(tpad, sps[0], sps[1], sps[2], sps[3], cp, un, p2a, wb, w2row, b2)


def kernel(x, edge_index, edge_attr, selection, batch, sel_w, sel_b,
           edge_mlp, node_mlp1, node_mlp2):
    N = x.shape[0]
    E = edge_index.shape[1]
    G = selection.shape[0]
    BN = 1024
    n_pad = ((N + 2047) // 2048) * 2048
    B = 1600

    (w1, b1), (w2, b2), (w3, b3), (w4, b4), (w5, b5) = edge_mlp
    (v1w, v1b1), (v2w, v2b2) = node_mlp1
    (n1w, n1b1), (n2w, n2b2) = node_mlp2
    F1 = w1.shape[1]

    p16 = jnp.concatenate([w1[0:9], jnp.zeros((7, F1), F32)], axis=0)
    q16 = jnp.concatenate([w1[9:18], jnp.zeros((7, F1), F32)], axis=0)
    w1c = w1[18:19]
    v1a = jnp.concatenate([v1w[0:9], jnp.zeros((7, v1w.shape[1]), F32)], axis=0)
    v1b = v1w[9:521]
    p2a = jnp.concatenate([n1w[0:9], jnp.zeros((7, n1w.shape[1]), F32)], axis=0)
    wb = n1w[9:521]
    w2row = n2w.reshape(1, -1)

    uw1, un = _prep(selection, sel_w, sel_b.reshape(1, -1),
                    w1[19:], b1.reshape(1, -1),
                    n1w[521:], n1b1.reshape(1, -1))

    tpad = jnp.zeros((n_pad, 16), F32)
    tpad = tpad.at[:N, 0:9].set(x)
    tpad = tpad.at[:N, 9].set(batch.astype(F32))

    row2d = edge_index[0].reshape(E // CH, CH)
    col2d = edge_index[1].reshape(E // CH, CH)

    ones_cst = jnp.ones((CH, 16), F32)
    zer_cst = jnp.zeros((n_pad // NS, 64), F32)
    zer16_cst = jnp.zeros((n_pad // NS, 16), F32)

    cp = _count(row2d, ones_cst, zer16_cst, E, n_pad)

    sizes = (38400, 41600, 38400, 41600)
    sps = []
    e0 = 0
    for sz in sizes:
        c0 = e0 // CH
        r2 = lax.slice_in_dim(row2d, c0, c0 + sz // CH, axis=0)
        c2 = lax.slice_in_dim(col2d, c0, c0 + sz // CH, axis=0)
        eah = lax.slice_in_dim(edge_attr, e0, e0 + sz, axis=0)
        gr, gc = _gather(tpad, r2, c2, sz)
        m = _edge(gr, gc, eah, uw1, p16, q16, w1c,
                  w2, b2.reshape(1, -1), w3, b3.reshape(1, -1),
                  w4, b4.reshape(1, -1), w5, b5.reshape(1, -1),
                  v1a, v1b, v1b1.reshape(1, -1), v2w, v2b2.reshape(1, -1), B)
        sps.append(_scatter(m, r2, zer_cst, sz, n_pad))
        e0 += sz

    outp = _node(tpad, sps, cp, un, p2a, wb, w2row,
                 n2b2.reshape(1, 1), BN)
    return outp[:N, 0]


# speedup vs baseline: 1.0063x; 1.0063x over previous
import functools

import jax
import jax.numpy as jnp
from jax import lax
from jax.experimental import pallas as pl
from jax.experimental.pallas import tpu as pltpu
from jax.experimental.pallas import tpu_sc as plsc

F32 = jnp.float32
I32 = jnp.int32
BF = jnp.bfloat16

NC = 2
NS = 16
NW = NC * NS
CH = 128


def _sc_mesh():
    return plsc.VectorSubcoreMesh(
        core_axis_name="c", subcore_axis_name="s", num_cores=NC, num_subcores=NS)


def _prep(selection, sel_w, sel_b, w1u, b1, wn2u, bn2):
    G = selection.shape[0]
    F1 = w1u.shape[1]
    F2 = wn2u.shape[1]

    def body(sel_ref, sw_ref, sb_ref, w1u_ref, b1_ref, wn2u_ref, bn2_ref,
             uw1_ref, un_ref):
        u = jnp.dot(sel_ref[...], sw_ref[...], preferred_element_type=F32)
        u = u + sb_ref[...]
        uw1_ref[...] = jnp.dot(u, w1u_ref[...], preferred_element_type=F32) + b1_ref[...]
        un_ref[...] = jnp.dot(u, wn2u_ref[...], preferred_element_type=F32) + bn2_ref[...]

    return pl.pallas_call(
        body,
        out_shape=(jax.ShapeDtypeStruct((G, F1), F32),
                   jax.ShapeDtypeStruct((G, F2), F32)),
    )(selection, sel_w, sel_b, w1u, b1, wn2u, bn2)


def _gather(tpad, row2d, col2d, E):
    nchunk = E // CH

    @functools.partial(
        pl.kernel,
        out_type=(jax.ShapeDtypeStruct((E, 16), F32),
                  jax.ShapeDtypeStruct((E, 16), F32)),
        mesh=_sc_mesh(),
        scratch_types=(
            pltpu.VMEM((CH,), I32),
            pltpu.VMEM((CH,), I32),
            pltpu.VMEM((CH, 16), F32),
            pltpu.VMEM((CH, 16), F32),
            pltpu.SemaphoreType.DMA,
            pltpu.SemaphoreType.DMA,
        ),
        compiler_params=pltpu.CompilerParams(use_tc_tiling_on_sc=False),
    )
    def k(t_hbm, r_hbm, c_hbm, gr_hbm, gc_hbm, ir_v, ic_v, br_v, bc_v, sr, sc_):
        cid = lax.axis_index("c")
        sid = lax.axis_index("s")
        wid = sid * NC + cid
        nloc = (nchunk - wid + NW - 1) // NW

        def step(j, carry):
            ch = wid + j * NW
            pltpu.sync_copy(r_hbm.at[ch], ir_v)
            pltpu.sync_copy(c_hbm.at[ch], ic_v)
            d1 = pltpu.async_copy(t_hbm.at[ir_v], br_v, sr)
            d2 = pltpu.async_copy(t_hbm.at[ic_v], bc_v, sc_)
            d1.wait()
            d2.wait()
            pltpu.sync_copy(br_v, gr_hbm.at[pl.ds(ch * CH, CH)])
            pltpu.sync_copy(bc_v, gc_hbm.at[pl.ds(ch * CH, CH)])
            return carry

        lax.fori_loop(0, nloc, step, 0)

    return k(tpad, row2d, col2d)


def _edge(gr, gc, ea, uw1, p16, q16, w1c, w2, b2, w3, b3, w4, b4, w5, b5,
          v1a, v1b, vb1, v2, vb2, B):
    E = gr.shape[0]
    nb = E // B
    G = uw1.shape[0]

    def body(gr_ref, gc_ref, ea_ref, uw1_ref, p_ref, q_ref, w1c_ref,
             w2_ref, b2_ref, w3_ref, b3_ref, w4_ref, b4_ref, w5_ref, b5_ref,
             v1a_ref, v1b_ref, vb1_ref, v2_ref, vb2_ref,
             o0_ref, o1_ref, o2_ref, o3_ref):
        HB = B // 2
        for s0 in (0, HB):
            g = gr_ref[pl.ds(s0, HB), :]
            gcv = gc_ref[pl.ds(s0, HB), :]
            ea = ea_ref[pl.ds(s0, HB), :]
            oh = (g[:, 9:10].astype(I32)
                  == lax.broadcasted_iota(I32, (HB, G), 1)).astype(F32)
            h = jnp.dot(g, p_ref[...], preferred_element_type=F32)
            h = h + jnp.dot(gcv, q_ref[...], preferred_element_type=F32)
            h = h + jnp.dot(oh, uw1_ref[...], preferred_element_type=F32)
            h = h + ea * w1c_ref[...]
            h = jnp.maximum(h, 0.0)
            h = jnp.maximum(jnp.dot(h, w2_ref[...], preferred_element_type=F32) + b2_ref[...], 0.0)
            h = jnp.maximum(jnp.dot(h, w3_ref[...], preferred_element_type=F32) + b3_ref[...], 0.0)
            h = jnp.maximum(jnp.dot(h, w4_ref[...], preferred_element_type=F32) + b4_ref[...], 0.0)
            e5 = jnp.dot(h, w5_ref[...], preferred_element_type=F32) + b5_ref[...]
            m = jnp.dot(gcv, v1a_ref[...], preferred_element_type=F32)
            m = m + jnp.dot(e5, v1b_ref[...], preferred_element_type=F32) + vb1_ref[...]
            m = jnp.maximum(m, 0.0)
            m = jnp.maximum(jnp.dot(m, v2_ref[...], preferred_element_type=F32) + vb2_ref[...], 0.0)
            o0_ref[pl.ds(s0, HB), :] = m[:, 0:128]
            o1_ref[pl.ds(s0, HB), :] = m[:, 128:256]
            o2_ref[pl.ds(s0, HB), :] = m[:, 256:384]
            o3_ref[pl.ds(s0, HB), :] = m[:, 384:512]

    def cst(*s):
        return pl.BlockSpec(s, lambda i: tuple(0 for _ in s))

    return pl.pallas_call(
        body,
        grid=(nb,),
        in_specs=[
            pl.BlockSpec((B, 16), lambda i: (i, 0)),
            pl.BlockSpec((B, 16), lambda i: (i, 0)),
            pl.BlockSpec((B, 1), lambda i: (i, 0)),
            cst(*uw1.shape), cst(*p16.shape), cst(*q16.shape), cst(*w1c.shape),
            cst(*w2.shape), cst(*b2.shape), cst(*w3.shape), cst(*b3.shape),
            cst(*w4.shape), cst(*b4.shape), cst(*w5.shape), cst(*b5.shape),
            cst(*v1a.shape), cst(*v1b.shape), cst(*vb1.shape),
            cst(*v2.shape), cst(*vb2.shape),
        ],
        out_specs=[pl.BlockSpec((B, 128), lambda i: (i, 0))] * 4,
        out_shape=[jax.ShapeDtypeStruct((E, 128), F32)] * 4,
    )(gr, gc, ea, uw1, p16, q16, w1c, w2, b2, w3, b3, w4, b4, w5, b5,
      v1a, v1b, vb1, v2, vb2)


def _scatter(m4, row2d, zer_cst, E, n_pad):
    nchunk = E // CH
    FC = 64
    stripe = n_pad // NS
    maxl = (nchunk + NW - 1) // NW

    @functools.partial(
        pl.kernel,
        out_type=jax.ShapeDtypeStruct((NC, n_pad, 512), F32),
        mesh=_sc_mesh(),
        scratch_types=(
            pltpu.VMEM((maxl, CH), I32),
            pltpu.VMEM((CH, FC), F32),
            pltpu.VMEM((CH, FC), F32),
            pltpu.VMEM((stripe, FC), F32),
            pltpu.VMEM_SHARED((n_pad, FC), F32),
            pltpu.SemaphoreType.DMA,
            pltpu.SemaphoreType.DMA,
        ),
        compiler_params=pltpu.CompilerParams(use_tc_tiling_on_sc=False),
    )
    def k(m0_hbm, m1_hbm, m2_hbm, m3_hbm, r_hbm, zer_hbm, sp_hbm,
          idx_v, mv0, mv1, zer_v, s_sh, sm0, sm1):
        cid = lax.axis_index("c")
        sid = lax.axis_index("s")
        wid = sid * NC + cid
        nloc = (nchunk - wid + NW - 1) // NW
        pltpu.sync_copy(zer_hbm, zer_v)

        def ld(j, carry):
            pltpu.sync_copy(r_hbm.at[wid + j * NW], idx_v.at[j])
            return carry

        lax.fori_loop(0, nloc, ld, 0)
        pltpu.sync_copy(zer_v, s_sh.at[pl.ds(sid * stripe, stripe)])
        plsc.subcore_barrier()

        mvs = (mv0, mv1)
        sms = (sm0, sm1)
        for p in range(8):
            src = (m0_hbm, m1_hbm, m2_hbm, m3_hbm)[p // 2]
            c0 = (p % 2) * FC
            f0 = p * FC

            def start(j, b, src=src, c0=c0):
                ch = wid + j * NW
                pltpu.async_copy(
                    src.at[pl.ds(ch * CH, CH), pl.ds(c0, FC)], mvs[b], sms[b])

            def drain(b, src=src, c0=c0):
                pltpu.make_async_copy(
                    src.at[pl.ds(0, CH), pl.ds(c0, FC)], mvs[b], sms[b]).wait()

            @pl.when(nloc > 0)
            def _():
                start(0, 0)

            def step(j, carry):
                even = lax.rem(j, 2) == 0

                @pl.when(even)
                def _():
                    drain(0)
                    @pl.when(j + 1 < nloc)
                    def _():
                        start(j + 1, 1)
                    pltpu.sync_copy(mv0, s_sh.at[idx_v.at[j]], add=True)

                @pl.when(jnp.logical_not(even))
                def _():
                    drain(1)
                    @pl.when(j + 1 < nloc)
                    def _():
                        start(j + 1, 0)
                    pltpu.sync_copy(mv1, s_sh.at[idx_v.at[j]], add=True)

                return carry

            lax.fori_loop(0, nloc, step, 0)
            plsc.subcore_barrier()
            pltpu.sync_copy(
                s_sh.at[pl.ds(sid * stripe, stripe)],
                sp_hbm.at[cid, pl.ds(sid * stripe, stripe), pl.ds(f0, FC)])
            if p < 7:
                pltpu.sync_copy(zer_v, s_sh.at[pl.ds(sid * stripe, stripe)])
            plsc.subcore_barrier()

    return k(m4[0], m4[1], m4[2], m4[3], row2d, zer_cst)


def _count(row2d, ones_cst, zer_cst, E, n_pad):
    nchunk = E // CH
    stripe = n_pad // NS

    @functools.partial(
        pl.kernel,
        out_type=jax.ShapeDtypeStruct((NC, n_pad, 16), F32),
        mesh=_sc_mesh(),
        scratch_types=(
            pltpu.VMEM((1, CH), I32),
            pltpu.VMEM((CH, 16), F32),
            pltpu.VMEM((stripe, 16), F32),
            pltpu.VMEM_SHARED((n_pad, 16), F32),
        ),
        compiler_params=pltpu.CompilerParams(use_tc_tiling_on_sc=False),
    )
    def k(r_hbm, ones_hbm, zer_hbm, cp_hbm, idx_v, ones_v, zer_v, c_sh):
        cid = lax.axis_index("c")
        sid = lax.axis_index("s")
        wid = sid * NC + cid
        nloc = (nchunk - wid + NW - 1) // NW
        pltpu.sync_copy(ones_hbm, ones_v)
        pltpu.sync_copy(zer_hbm, zer_v)
        pltpu.sync_copy(zer_v, c_sh.at[pl.ds(sid * stripe, stripe)])
        plsc.subcore_barrier()

        def step(j, carry):
            ch = wid + j * NW
            pltpu.sync_copy(r_hbm.at[ch], idx_v.at[0])
            pltpu.sync_copy(ones_v, c_sh.at[idx_v.at[0]], add=True)
            return carry

        lax.fori_loop(0, nloc, step, 0)
        plsc.subcore_barrier()
        pltpu.sync_copy(c_sh.at[pl.ds(sid * stripe, stripe)],
                        cp_hbm.at[cid, pl.ds(sid * stripe, stripe)])

    return k(row2d, ones_cst, zer_cst)


def _node(tpad, sps, cp, un, p2a, wb, w2row, b2, BN):
    n_pad = tpad.shape[0]
    nb = n_pad // BN
    G = un.shape[0]

    def body(t_ref, s0_ref, s1_ref, cp_ref, un_ref, p2_ref,
             wb_ref, w2_ref, b2_ref, o_ref):
        t = t_ref[...]
        cpv = cp_ref[...]
        s = None
        for r in (s0_ref, s1_ref):
            v = r[...]
            vv = v[0] + v[1]
            s = vv if s is None else s + vv
        cnt = cpv[0][:, 0:1] + cpv[1][:, 0:1]
        mean = s / jnp.maximum(cnt, 1.0)
        oh = (t[:, 9:10].astype(I32)
              == lax.broadcasted_iota(I32, (BN, G), 1)).astype(F32)
        l1 = jnp.dot(t, p2_ref[...], preferred_element_type=F32)
        l1 = l1 + jnp.dot(mean, wb_ref[...], preferred_element_type=F32)
        l1 = l1 + jnp.dot(oh, un_ref[...], preferred_element_type=F32)
        l1 = jnp.maximum(l1, 0.0)
        o_ref[...] = jnp.sum(l1 * w2_ref[...], axis=1, keepdims=True) + b2_ref[...]

    def cst(*s):
        return pl.BlockSpec(s, lambda i: tuple(0 for _ in s))

    return pl.pallas_call(
        body,
        grid=(nb,),
        in_specs=[
            pl.BlockSpec((BN, 16), lambda i: (i, 0)),
            pl.BlockSpec((2, BN, 512), lambda i: (0, i, 0)),
            pl.BlockSpec((2, BN, 512), lambda i: (0, i, 0)),
            pl.BlockSpec((2, BN, 16), lambda i: (0, i, 0)),
            cst(*un.shape), cst(*p2a.shape), cst(*wb.shape),
            cst(*w2row.shape), cst(*b2.shape),
        ],
        out_specs=pl.BlockSpec((BN, 1), lambda i: (i, 0)),
        out_shape=jax.ShapeDtypeStruct((n_pad, 1), F32),
    )(tpad, sps[0], sps[1], cp, un, p2a, wb, w2row, b2)


def kernel(x, edge_index, edge_attr, selection, batch, sel_w, sel_b,
           edge_mlp, node_mlp1, node_mlp2):
    N = x.shape[0]
    E = edge_index.shape[1]
    G = selection.shape[0]
    BN = 1024
    n_pad = ((N + 2047) // 2048) * 2048
    B = 1600

    (w1, b1), (w2, b2), (w3, b3), (w4, b4), (w5, b5) = edge_mlp
    (v1w, v1b1), (v2w, v2b2) = node_mlp1
    (n1w, n1b1), (n2w, n2b2) = node_mlp2
    F1 = w1.shape[1]

    p16 = jnp.concatenate([w1[0:9], jnp.zeros((7, F1), F32)], axis=0)
    q16 = jnp.concatenate([w1[9:18], jnp.zeros((7, F1), F32)], axis=0)
    w1c = w1[18:19]
    v1a = jnp.concatenate([v1w[0:9], jnp.zeros((7, v1w.shape[1]), F32)], axis=0)
    v1b = v1w[9:521]
    p2a = jnp.concatenate([n1w[0:9], jnp.zeros((7, n1w.shape[1]), F32)], axis=0)
    wb = n1w[9:521]
    w2row = n2w.reshape(1, -1)

    uw1, un = _prep(selection, sel_w, sel_b.reshape(1, -1),
                    w1[19:], b1.reshape(1, -1),
                    n1w[521:], n1b1.reshape(1, -1))

    tpad = jnp.zeros((n_pad, 16), F32)
    tpad = tpad.at[:N, 0:9].set(x)
    tpad = tpad.at[:N, 9].set(batch.astype(F32))

    row2d = edge_index[0].reshape(E // CH, CH)
    col2d = edge_index[1].reshape(E // CH, CH)

    ones_cst = jnp.ones((CH, 16), F32)
    zer_cst = jnp.zeros((n_pad // NS, 64), F32)
    zer16_cst = jnp.zeros((n_pad // NS, 16), F32)

    cp = _count(row2d, ones_cst, zer16_cst, E, n_pad)

    sizes = (80000, 80000)
    sps = []
    e0 = 0
    for sz in sizes:
        c0 = e0 // CH
        r2 = lax.slice_in_dim(row2d, c0, c0 + sz // CH, axis=0)
        c2 = lax.slice_in_dim(col2d, c0, c0 + sz // CH, axis=0)
        eah = lax.slice_in_dim(edge_attr, e0, e0 + sz, axis=0)
        gr, gc = _gather(tpad, r2, c2, sz)
        m = _edge(gr, gc, eah, uw1, p16, q16, w1c,
                  w2, b2.reshape(1, -1), w3, b3.reshape(1, -1),
                  w4, b4.reshape(1, -1), w5, b5.reshape(1, -1),
                  v1a, v1b, v1b1.reshape(1, -1), v2w, v2b2.reshape(1, -1), B)
        sps.append(_scatter(m, r2, zer_cst, sz, n_pad))
        e0 += sz

    outp = _node(tpad, sps, cp, un, p2a, wb, w2row,
                 n2b2.reshape(1, 1), BN)
    return outp[:N, 0]
